# baseline (speedup 1.0000x reference)
import jax
import jax.numpy as jnp
from jax import lax
from jax.experimental import pallas as pl
from jax.experimental.pallas import tpu as pltpu
from jax.experimental.pallas import tpu_sc as plsc

H, W = 512, 512
ROWS = 4 * 96 * H
NW = 32
PER_W = ROWS // NW
CHUNK = 64
NVR = W // 16


def _sc_body(x_hbm, o_hbm, buf):
    c = lax.axis_index("c")
    s = lax.axis_index("s")
    wid = s * 2 + c
    base = wid * PER_W

    def chunk_body(k, carry):
        row0 = base + k * CHUNK
        pltpu.sync_copy(x_hbm.at[pl.ds(row0, CHUNK)], buf)

        def odd_body(i, carry2):
            r = 2 * i + 1
            vals = [buf[r, pl.ds(16 * j, 16)] for j in range(NVR)]
            for j in range(NVR):
                buf[r, pl.ds(16 * (NVR - 1 - j), 16)] = lax.rev(vals[j], (0,))
            return carry2

        lax.fori_loop(0, CHUNK // 2, odd_body, 0)
        pltpu.sync_copy(buf, o_hbm.at[pl.ds(row0, CHUNK)])
        return carry

    lax.fori_loop(0, PER_W // CHUNK, chunk_body, 0)


def kernel(x, idx):
    B, C, Hh, Ww = x.shape
    rows = B * C * Hh
    x2 = x.reshape(rows, Ww)
    mesh = plsc.VectorSubcoreMesh(core_axis_name="c", subcore_axis_name="s")
    out = pl.kernel(
        _sc_body,
        out_type=jax.ShapeDtypeStruct((rows, Ww), x.dtype),
        mesh=mesh,
        scratch_types=[pltpu.VMEM((CHUNK, Ww), jnp.float32)],
    )(x2)
    return out.reshape(B, C, Hh * Ww)


# speedup vs baseline: 3.0094x; 3.0094x over previous
import jax
import jax.numpy as jnp
from jax.experimental import pallas as pl
from jax.experimental.pallas import tpu as pltpu

H, W = 512, 512
CB = 8
HS = 2
H2 = H // HS


def _snake_block(x_ref, o_ref):
    y = x_ref[0].reshape(CB * H2, W)
    n = y.shape[0]
    ridx = 127 - jax.lax.broadcasted_iota(jnp.int32, (n, 128), 1)
    chunks = [
        jnp.take_along_axis(y[:, W - 128 * (j + 1):W - 128 * j], ridx, axis=1)
        for j in range(4)
    ]
    rev = jnp.concatenate(chunks, axis=1)
    r = jax.lax.broadcasted_iota(jnp.int32, y.shape, 0)
    sel = jnp.where((r % 2) == 0, y, rev)
    o_ref[...] = sel.reshape(CB, H2 * W)


def kernel(x, idx):
    B, C, Hh, Ww = x.shape
    nblk = B * C // CB
    cblk = C // CB
    out = pl.pallas_call(
        _snake_block,
        out_shape=jax.ShapeDtypeStruct((B * C, Hh * Ww), x.dtype),
        grid=(nblk, HS),
        in_specs=[pl.BlockSpec((1, CB, H2, Ww),
                               lambda g, h: (g // cblk, g % cblk, h, 0))],
        out_specs=pl.BlockSpec((CB, H2 * Ww), lambda g, h: (g, h)),
        compiler_params=pltpu.CompilerParams(
            dimension_semantics=("parallel", "arbitrary"),
        ),
    )(x)
    return out.reshape(B, C, Hh * Ww)


# speedup vs baseline: 3.3261x; 1.1052x over previous
import jax
import jax.numpy as jnp
from jax.experimental import pallas as pl
from jax.experimental.pallas import tpu as pltpu

H, W = 512, 512
CB = 8


def _snake_block(x_ref, o_ref):
    y = x_ref[0].reshape(CB * H, W)
    n = y.shape[0]
    ridx = 127 - jax.lax.broadcasted_iota(jnp.int32, (n, 128), 1)
    chunks = [
        jnp.take_along_axis(y[:, W - 128 * (j + 1):W - 128 * j], ridx, axis=1)
        for j in range(4)
    ]
    rev = jnp.concatenate(chunks, axis=1)
    r = jax.lax.broadcasted_iota(jnp.int32, y.shape, 0)
    sel = jnp.where((r % 2) == 0, y, rev)
    o_ref[...] = sel.reshape(CB, H * W)


def kernel(x, idx):
    B, C, Hh, Ww = x.shape
    nblk = B * C // CB
    cblk = C // CB
    out = pl.pallas_call(
        _snake_block,
        out_shape=jax.ShapeDtypeStruct((B * C, Hh * Ww), x.dtype),
        grid=(nblk,),
        in_specs=[pl.BlockSpec((1, CB, Hh, Ww),
                               lambda g: (g // cblk, g % cblk, 0, 0))],
        out_specs=pl.BlockSpec((CB, Hh * Ww), lambda g: (g, 0)),
        compiler_params=pltpu.CompilerParams(
            dimension_semantics=("parallel",),
        ),
    )(x)
    return out.reshape(B, C, Hh * Ww)
